# baseline (speedup 1.0000x reference)
import functools
import math

import jax
import jax.numpy as jnp
from jax import lax
from jax.experimental import pallas as pl
from jax.experimental.pallas import tpu as pltpu
from jax.experimental.pallas import tpu_sc as plsc

NC = 2
NS = 16
NW = NC * NS
BLK = 128
ROWG = 10
DEPTH = 4
EUN = 4


def _build(b, l, vocab, emb, scale):
    mesh = plsc.VectorSubcoreMesh(
        core_axis_name="c", subcore_axis_name="s", num_cores=NC, num_subcores=NS
    )
    n_groups = l // ROWG

    @functools.partial(
        pl.kernel,
        mesh=mesh,
        out_type=jax.ShapeDtypeStruct((l, emb, b), jnp.float32),
        compiler_params=pltpu.CompilerParams(needs_layout_passes=False),
        scratch_types=[
            pltpu.VMEM((l, BLK), jnp.int32),
            pltpu.VMEM((DEPTH, BLK, 128), jnp.float32),
            pltpu.VMEM((DEPTH, BLK), jnp.int32),
            pltpu.VMEM((2, emb, BLK), jnp.float32),
            pltpu.SemaphoreType.DMA,
            pltpu.SemaphoreType.DMA,
        ],
    )
    def k(tokt_hbm, pair_hbm, out_hbm, idx_v, rows_v, pidx_v, obuf_v, gsem, osem):
        wid = lax.axis_index("s") * NC + lax.axis_index("c")
        b0 = pl.multiple_of(wid * BLK, BLK)
        pltpu.sync_copy(tokt_hbm.at[:, pl.ds(b0, BLK)], idx_v)

        jiota = lax.iota(jnp.int32, 16)

        def fire_gathers(r, buf):
            for jg in range(BLK // 16):
                iv = idx_v[r, pl.ds(jg * 16, 16)]
                pidx_v[buf, pl.ds(jg * 16, 16)] = lax.shift_right_logical(iv, 1)
            return [
                pltpu.async_copy(
                    pair_hbm.at[pidx_v.at[buf]],
                    rows_v.at[buf],
                    gsem,
                )
            ]

        def transpose_scale(r, buf, obuf):
            rowvs = [jiota + (jg * 16) for jg in range(BLK // 16)]
            cols0 = tuple(
                lax.bitwise_and(idx_v[r, pl.ds(jg * 16, 16)], 1) * 64
                for jg in range(BLK // 16)
            )

            def e_body(e4, cols):
                cols = list(cols)
                for k in range(EUN):
                    e = e4 * EUN + k
                    for jg in range(BLK // 16):
                        v = plsc.load_gather(rows_v.at[buf], [rowvs[jg], cols[jg]])
                        obuf_v[obuf, e, pl.ds(jg * 16, 16)] = v * scale
                        cols[jg] = cols[jg] + 1
                return tuple(cols)

            lax.fori_loop(0, emb // EUN, e_body, cols0)

        def group_body(g, carry):
            lbase = g * ROWG
            gh = {}
            for k in range(DEPTH - 1):
                gh[k] = fire_gathers(lbase + k, k)
            oh = {}
            for r in range(ROWG):
                lr = lbase + r
                buf = r % DEPTH
                ob = r % 2
                for h in gh.pop(buf):
                    h.wait()
                if r + DEPTH - 1 < ROWG:
                    nb = (r + DEPTH - 1) % DEPTH
                    gh[nb] = fire_gathers(lr + DEPTH - 1, nb)
                if r - 2 in oh:
                    oh.pop(r - 2).wait()
                transpose_scale(lr, buf, ob)
                oh[r] = pltpu.async_copy(
                    obuf_v.at[ob],
                    out_hbm.at[lr].at[:, pl.ds(b0, BLK)],
                    osem,
                )
            oh.pop(ROWG - 2).wait()
            oh.pop(ROWG - 1).wait()
            return carry

        lax.fori_loop(0, n_groups, group_body, 0)

    return k


def kernel(tokens, table):
    b, l = tokens.shape
    vocab, emb = table.shape
    scale = math.sqrt(emb)
    tokt = jnp.swapaxes(tokens.astype(jnp.int32), 0, 1)
    pair = table.reshape(vocab // 2, 2 * emb)
    outt = _build(b, l, vocab, emb, scale)(tokt, pair)
    return jnp.transpose(outt, (2, 0, 1))


# speedup vs baseline: 1.1161x; 1.1161x over previous
import functools
import math

import jax
import jax.numpy as jnp
from jax import lax
from jax.experimental import pallas as pl
from jax.experimental.pallas import tpu as pltpu
from jax.experimental.pallas import tpu_sc as plsc

NC = 2
NS = 16
NW = NC * NS
BLK = 128
ROWG = 10
DEPTH = 4
EUN = 4


def _build(b, l, vocab, emb, scale):
    mesh = plsc.VectorSubcoreMesh(
        core_axis_name="c", subcore_axis_name="s", num_cores=NC, num_subcores=NS
    )
    n_groups = l // ROWG

    @functools.partial(
        pl.kernel,
        mesh=mesh,
        out_type=jax.ShapeDtypeStruct((l, emb, b), jnp.float32),
        compiler_params=pltpu.CompilerParams(needs_layout_passes=False),
        scratch_types=[
            pltpu.VMEM((l, BLK), jnp.int32),
            pltpu.VMEM((DEPTH, BLK, 128), jnp.float32),
            pltpu.VMEM((DEPTH, BLK), jnp.int32),
            pltpu.VMEM((2, emb, BLK), jnp.float32),
            pltpu.SemaphoreType.DMA,
            pltpu.SemaphoreType.DMA,
        ],
    )
    def k(tokt_hbm, pair_hbm, out_hbm, idx_v, rows_v, pidx_v, obuf_v, gsem, osem):
        wid = lax.axis_index("s") * NC + lax.axis_index("c")
        b0 = pl.multiple_of(wid * BLK, BLK)
        pltpu.sync_copy(tokt_hbm.at[:, pl.ds(b0, BLK)], idx_v)

        jiota = lax.iota(jnp.int32, 16)

        def fire_gathers(r, buf):
            for jg in range(BLK // 16):
                iv = idx_v[r, pl.ds(jg * 16, 16)]
                pidx_v[buf, pl.ds(jg * 16, 16)] = lax.shift_right_logical(iv, 1)
            return [
                pltpu.async_copy(
                    pair_hbm.at[pidx_v.at[buf]],
                    rows_v.at[buf],
                    gsem,
                )
            ]

        evecs = [jiota + (e16 * 16) for e16 in range(emb // 16)]

        def transpose_scale(r, buf, obuf):
            def jg_body(jg, carry):
                tv = idx_v[r, pl.ds(jg * 16, 16)]
                for jj in range(16):
                    j = jg * 16 + jj
                    h64 = lax.bitwise_and(tv[jj], 1) * 64
                    bvec = jnp.full((16,), j, jnp.int32)
                    for e16 in range(emb // 16):
                        v = rows_v[buf, j, pl.ds(h64 + e16 * 16, 16)]
                        plsc.store_scatter(
                            obuf_v.at[obuf], [evecs[e16], bvec], v * scale
                        )
                return carry

            lax.fori_loop(0, BLK // 16, jg_body, 0)

        def group_body(g, carry):
            lbase = g * ROWG
            gh = {}
            for k in range(DEPTH - 1):
                gh[k] = fire_gathers(lbase + k, k)
            oh = {}
            for r in range(ROWG):
                lr = lbase + r
                buf = r % DEPTH
                ob = r % 2
                for h in gh.pop(buf):
                    h.wait()
                if r + DEPTH - 1 < ROWG:
                    nb = (r + DEPTH - 1) % DEPTH
                    gh[nb] = fire_gathers(lr + DEPTH - 1, nb)
                if r - 2 in oh:
                    oh.pop(r - 2).wait()
                transpose_scale(lr, buf, ob)
                oh[r] = pltpu.async_copy(
                    obuf_v.at[ob],
                    out_hbm.at[lr].at[:, pl.ds(b0, BLK)],
                    osem,
                )
            oh.pop(ROWG - 2).wait()
            oh.pop(ROWG - 1).wait()
            return carry

        lax.fori_loop(0, n_groups, group_body, 0)

    return k


def kernel(tokens, table):
    b, l = tokens.shape
    vocab, emb = table.shape
    scale = math.sqrt(emb)
    tokt = jnp.swapaxes(tokens.astype(jnp.int32), 0, 1)
    pair = table.reshape(vocab // 2, 2 * emb)
    outt = _build(b, l, vocab, emb, scale)(tokt, pair)
    return jnp.transpose(outt, (2, 0, 1))


# speedup vs baseline: 1.5967x; 1.4306x over previous
import functools
import math

import jax
import jax.numpy as jnp
from jax import lax
from jax.experimental import pallas as pl
from jax.experimental.pallas import tpu as pltpu
from jax.experimental.pallas import tpu_sc as plsc

NC = 2
NS = 16
NW = NC * NS
IDX_ROW = 128
CHUNK_ROWS = 4
UNROLL = 8


def _build(n_rows, vocab, emb, scale):
    chunk = CHUNK_ROWS * IDX_ROW
    rows_per_w = n_rows // NW
    n_chunks = rows_per_w // CHUNK_ROWS
    mesh = plsc.VectorSubcoreMesh(
        core_axis_name="c", subcore_axis_name="s", num_cores=NC, num_subcores=NS
    )

    @functools.partial(
        pl.kernel,
        mesh=mesh,
        out_type=jax.ShapeDtypeStruct((n_rows * IDX_ROW, emb), jnp.float32),
        compiler_params=pltpu.CompilerParams(use_tc_tiling_on_sc=False),
        scratch_types=[
            pltpu.VMEM((rows_per_w, IDX_ROW), jnp.int32),
            pltpu.VMEM((chunk, emb), jnp.float32),
            pltpu.SemaphoreType.DMA,
            pltpu.SemaphoreType.DMA,
        ],
    )
    def k(tok_hbm, table_hbm, out_hbm, idx_v, rows_v, gsem, osem):
        wid = lax.axis_index("s") * NC + lax.axis_index("c")
        row0 = wid * rows_per_w
        pltpu.sync_copy(tok_hbm.at[pl.ds(row0, rows_per_w)], idx_v)

        def chunk_body(g, carry):
            base = (row0 + g * CHUNK_ROWS) * IDX_ROW
            gathers = []
            for j in range(CHUNK_ROWS):
                gathers.append(
                    pltpu.async_copy(
                        table_hbm.at[idx_v.at[g * CHUNK_ROWS + j]],
                        rows_v.at[pl.ds(j * IDX_ROW, IDX_ROW)],
                        gsem,
                    )
                )
            outs = []
            for j in range(CHUNK_ROWS):
                gathers[j].wait()

                def scale_body(i, c2, j=j):
                    for rr in range(UNROLL):
                        r = j * IDX_ROW + i * UNROLL + rr
                        for c in range(emb // 16):
                            v = rows_v[r, pl.ds(c * 16, 16)]
                            rows_v[r, pl.ds(c * 16, 16)] = v * scale
                    return c2

                lax.fori_loop(0, IDX_ROW // UNROLL, scale_body, 0)
                outs.append(
                    pltpu.async_copy(
                        rows_v.at[pl.ds(j * IDX_ROW, IDX_ROW)],
                        out_hbm.at[pl.ds(base + j * IDX_ROW, IDX_ROW)],
                        osem,
                    )
                )
            for o in outs:
                o.wait()
            return carry

        lax.fori_loop(0, n_chunks, chunk_body, 0)

    return k


def kernel(tokens, table):
    b, l = tokens.shape
    vocab, emb = table.shape
    n = b * l
    n_rows = n // IDX_ROW
    scale = math.sqrt(emb)
    tok = tokens.astype(jnp.int32).reshape(n_rows, IDX_ROW)
    out = _build(n_rows, vocab, emb, scale)(tok, table)
    return out.reshape(b, l, emb)


# speedup vs baseline: 1.6113x; 1.0091x over previous
import functools
import math

import jax
import jax.numpy as jnp
from jax import lax
from jax.experimental import pallas as pl
from jax.experimental.pallas import tpu as pltpu
from jax.experimental.pallas import tpu_sc as plsc

NC = 2
NS = 16
NW = NC * NS
IDX_ROW = 128
CHUNK_ROWS = 8
UNROLL = 8


def _build(n_rows, vocab, emb, scale):
    chunk = CHUNK_ROWS * IDX_ROW
    rows_per_w = n_rows // NW
    n_chunks = rows_per_w // CHUNK_ROWS
    mesh = plsc.VectorSubcoreMesh(
        core_axis_name="c", subcore_axis_name="s", num_cores=NC, num_subcores=NS
    )

    @functools.partial(
        pl.kernel,
        mesh=mesh,
        out_type=jax.ShapeDtypeStruct((n_rows * IDX_ROW, emb), jnp.float32),
        compiler_params=pltpu.CompilerParams(use_tc_tiling_on_sc=False),
        scratch_types=[
            pltpu.VMEM((rows_per_w, IDX_ROW), jnp.int32),
            pltpu.VMEM((chunk, emb), jnp.float32),
            pltpu.SemaphoreType.DMA,
            pltpu.SemaphoreType.DMA,
        ],
    )
    def k(tok_hbm, table_hbm, out_hbm, idx_v, rows_v, gsem, osem):
        wid = lax.axis_index("s") * NC + lax.axis_index("c")
        row0 = wid * rows_per_w
        pltpu.sync_copy(tok_hbm.at[pl.ds(row0, rows_per_w)], idx_v)

        def chunk_body(g, carry):
            base = (row0 + g * CHUNK_ROWS) * IDX_ROW
            gathers = []
            for j in range(CHUNK_ROWS):
                gathers.append(
                    pltpu.async_copy(
                        table_hbm.at[idx_v.at[g * CHUNK_ROWS + j]],
                        rows_v.at[pl.ds(j * IDX_ROW, IDX_ROW)],
                        gsem,
                    )
                )
            outs = []
            for j in range(CHUNK_ROWS):
                gathers[j].wait()

                def scale_body(i, c2, j=j):
                    for rr in range(UNROLL):
                        r = j * IDX_ROW + i * UNROLL + rr
                        for c in range(emb // 16):
                            v = rows_v[r, pl.ds(c * 16, 16)]
                            rows_v[r, pl.ds(c * 16, 16)] = v * scale
                    return c2

                lax.fori_loop(0, IDX_ROW // UNROLL, scale_body, 0)
                outs.append(
                    pltpu.async_copy(
                        rows_v.at[pl.ds(j * IDX_ROW, IDX_ROW)],
                        out_hbm.at[pl.ds(base + j * IDX_ROW, IDX_ROW)],
                        osem,
                    )
                )
            for o in outs:
                o.wait()
            return carry

        lax.fori_loop(0, n_chunks, chunk_body, 0)

    return k


def kernel(tokens, table):
    b, l = tokens.shape
    vocab, emb = table.shape
    n = b * l
    n_rows = n // IDX_ROW
    scale = math.sqrt(emb)
    tok = tokens.astype(jnp.int32).reshape(n_rows, IDX_ROW)
    out = _build(n_rows, vocab, emb, scale)(tok, table)
    return out.reshape(b, l, emb)
